# baseline (speedup 1.0000x reference)
import functools

import jax
import jax.numpy as jnp
from jax import lax
from jax.experimental import pallas as pl
from jax.experimental.pallas import tpu as pltpu
from jax.experimental.pallas import tpu_sc as plsc

VOCAB = 1000000
DIM = 16
NEG = 20
B = 4096
N = B * (1 + NEG)
NW = 32
BPW = B // NW
RPW = N // NW
LANE = 128
NCH = RPW // LANE
IDXROWS = 24
ROWS2D = N // LANE
POS_ROWS = B // LANE

_mesh = plsc.VectorSubcoreMesh(core_axis_name="c", subcore_axis_name="s")


@functools.partial(
    pl.kernel,
    out_type=jax.ShapeDtypeStruct((N,), jnp.float32),
    mesh=_mesh,
    scratch_types=[
        pltpu.VMEM((IDXROWS, LANE), jnp.int32),
        pltpu.VMEM((1, LANE), jnp.int32),
        pltpu.VMEM((IDXROWS, LANE), jnp.int32),
        pltpu.VMEM((1, LANE), jnp.int32),
        pltpu.VMEM((LANE, LANE), jnp.float32),
        pltpu.VMEM((LANE, LANE), jnp.float32),
        pltpu.VMEM((LANE, LANE), jnp.float32),
        pltpu.VMEM((DIM, LANE), jnp.float32),
        pltpu.VMEM((RPW,), jnp.float32),
        pltpu.SemaphoreType.DMA,
        pltpu.SemaphoreType.DMA,
    ],
    compiler_params=pltpu.CompilerParams(needs_layout_passes=False),
)
def _sc_scores(emb_u, emb_v, ug, vg, us, vs, out,
               ug_v, vg_v, usub_v, vsub_v, ubufa, ubufb, vbuf, cent,
               scores_v, usem, vsem):
    wid = lax.axis_index("s") * 2 + lax.axis_index("c")

    pltpu.sync_copy(ug.at[wid], ug_v)
    pltpu.sync_copy(vg.at[wid], vg_v)
    pltpu.sync_copy(us.at[wid], usub_v)
    pltpu.sync_copy(vs.at[wid], vsub_v)

    iota16 = lax.iota(jnp.int32, 16)

    pltpu.async_copy(emb_v.at[vg_v.at[0]], vbuf, vsem).wait()
    pltpu.async_copy(emb_u.at[ug_v.at[0]], ubufa, usem)

    def v_extract(ph, carry):
        svec = vsub_v[0, pl.ds(ph * 16, 16)]
        slot = ph * 16 + iota16
        for d in range(DIM):
            cent[d, pl.ds(ph * 16, 16)] = plsc.load_gather(
                vbuf, [slot, svec + d])
        return carry

    lax.fori_loop(0, BPW // 16, v_extract, 0)

    def score_chunk(j, buf):
        def blk(b, carry):
            row = j * LANE + b * 16
            svec = usub_v[row // LANE, pl.ds(row % LANE, 16)]
            slot = b * 16 + iota16
            rvec = row + iota16
            bvec = jnp.where(rvec < BPW, rvec, (rvec - BPW) // NEG)
            acc = jnp.zeros((16,), jnp.float32)
            for d in range(DIM):
                uc = plsc.load_gather(buf, [slot, svec + d])
                vc = plsc.load_gather(cent, [jnp.full((16,), d, jnp.int32), bvec])
                acc = acc + uc * vc
            scores_v[pl.ds(row, 16)] = acc
            return carry

        lax.fori_loop(0, LANE // 16, blk, 0)

    def u_chunk(j, carry):
        even = j % 2 == 0

        @pl.when(jnp.logical_and(j + 1 < NCH, even))
        def _():
            pltpu.async_copy(emb_u.at[ug_v.at[j + 1]], ubufb, usem)

        @pl.when(jnp.logical_and(j + 1 < NCH, jnp.logical_not(even)))
        def _():
            pltpu.async_copy(emb_u.at[ug_v.at[j + 1]], ubufa, usem)

        @pl.when(even)
        def _():
            pltpu.make_async_copy(emb_u.at[ug_v.at[j]], ubufa, usem).wait()
            score_chunk(j, ubufa)

        @pl.when(jnp.logical_not(even))
        def _():
            pltpu.make_async_copy(emb_u.at[ug_v.at[j]], ubufb, usem).wait()
            score_chunk(j, ubufb)

        return carry

    lax.fori_loop(0, NCH, u_chunk, 0)

    pltpu.sync_copy(scores_v.at[pl.ds(0, BPW)], out.at[pl.ds(wid * BPW, BPW)])
    pltpu.sync_copy(scores_v.at[pl.ds(BPW, RPW - BPW)],
                    out.at[pl.ds(B + wid * (RPW - BPW), RPW - BPW)])


def _tc_reduce_body(s_ref, o_ref):
    s = s_ref[...]
    ridx = lax.broadcasted_iota(jnp.int32, (ROWS2D, LANE), 0)
    t = jnp.where(ridx < POS_ROWS, s, -s)
    ls = jnp.minimum(t, 0.0) - jnp.log1p(jnp.exp(-jnp.abs(t)))
    o_ref[0, 0] = -jnp.sum(ls) / B


_tc_reduce = pl.pallas_call(
    _tc_reduce_body,
    out_shape=jax.ShapeDtypeStruct((1, 1), jnp.float32),
    out_specs=pl.BlockSpec(memory_space=pltpu.SMEM),
)


def _pad_rows(x2d, rows):
    out = jnp.zeros((NW, rows * LANE), x2d.dtype)
    out = lax.dynamic_update_slice(out, x2d, (0, 0))
    return out.reshape(NW, rows, LANE)


def kernel(embedding_v, embedding_u, center_words, target_words, negative_words):
    c = center_words.reshape(-1).astype(jnp.int32)
    t = target_words.reshape(-1).astype(jnp.int32)
    n = negative_words.reshape(-1).astype(jnp.int32)
    uidx = jnp.concatenate(
        [t.reshape(NW, BPW), n.reshape(NW, RPW - BPW)], axis=1)
    ug = _pad_rows(uidx // 8, IDXROWS)
    us = _pad_rows((uidx % 8) * DIM, IDXROWS)
    vidx = c.reshape(NW, BPW)
    vg = (vidx // 8).reshape(NW, 1, LANE)
    vs = ((vidx % 8) * DIM).reshape(NW, 1, LANE)
    emb_u2 = embedding_u.reshape(VOCAB // 8, 8 * DIM)
    emb_v2 = embedding_v.reshape(VOCAB // 8, 8 * DIM)
    scores = _sc_scores(emb_u2, emb_v2, ug, vg, us, vs)
    loss = _tc_reduce(scores.reshape(ROWS2D, LANE))
    return loss[0, 0]


# speedup vs baseline: 1.6473x; 1.6473x over previous
import functools

import jax
import jax.numpy as jnp
from jax import lax
from jax.experimental import pallas as pl
from jax.experimental.pallas import tpu as pltpu
from jax.experimental.pallas import tpu_sc as plsc

VOCAB = 1000000
DIM = 16
NEG = 20
B = 4096
N = B * (1 + NEG)
NW = 32
BPW = B // NW
RPW = N // NW
LANE = 128
NCH = RPW // LANE
IDXROWS = 24
ROWS2D = N // LANE
POS_ROWS = B // LANE
CSZ = 2048
TB = VOCAB // LANE * LANE
BASEMAX = TB - CSZ
CPW = B // NW

_mesh = plsc.VectorSubcoreMesh(core_axis_name="c", subcore_axis_name="s")


def _window_base(cols, done, base):
    mincol = jnp.min(jnp.where(done, VOCAB, cols))
    inwin = jnp.logical_and(mincol >= base, mincol < base + CSZ)
    nb = jnp.minimum((mincol // LANE) * LANE, BASEMAX)
    return jnp.where(inwin, base, nb)


@functools.partial(
    pl.kernel,
    out_type=jax.ShapeDtypeStruct((B // 8, LANE), jnp.float32),
    mesh=_mesh,
    scratch_types=[
        pltpu.VMEM((1, LANE), jnp.int32),
        pltpu.VMEM((DIM, CSZ), jnp.float32),
        pltpu.VMEM((DIM, 64), jnp.float32),
        pltpu.VMEM((16, LANE), jnp.float32),
    ],
    compiler_params=pltpu.CompilerParams(needs_layout_passes=False),
)
def _build_cent(emb_t, tail, csort, out, cs_v, chunk_v, tail_v, stage_v):
    wid = lax.axis_index("s") * 2 + lax.axis_index("c")
    pltpu.sync_copy(csort.at[wid], cs_v)
    pltpu.sync_copy(tail, tail_v)
    iota16 = lax.iota(jnp.int32, 16)

    def blk(b, base):
        cols = cs_v[0, pl.ds(b * 16, 16)]
        pos = b * 16 + iota16
        rowv = pos // 8
        lanev = (pos % 8) * DIM
        tmask = cols >= TB

        def extract(src, colloc, mask):
            for d in range(DIM):
                vals = plsc.load_gather(
                    src, [jnp.full((16,), d, jnp.int32), colloc])
                plsc.store_scatter(stage_v, [rowv, lanev + d], vals, mask=mask)

        @pl.when(jnp.any(tmask))
        def _():
            extract(tail_v, jnp.clip(cols - TB, 0, 63), tmask)

        def cond(st):
            return jnp.logical_not(jnp.all(st[0]))

        def body(st):
            done, base = st
            nb = _window_base(cols, done, base)

            @pl.when(nb != base)
            def _():
                pltpu.sync_copy(
                    emb_t.at[:, pl.ds(pl.multiple_of(nb, LANE), CSZ)], chunk_v)

            mask = jnp.logical_and(
                jnp.logical_not(done),
                jnp.logical_and(cols >= nb, cols < nb + CSZ))
            extract(chunk_v, jnp.clip(cols - nb, 0, CSZ - 1), mask)
            return jnp.logical_or(done, mask), nb

        _, base = lax.while_loop(cond, body, (tmask, base))
        return base

    lax.fori_loop(0, CPW // 16, blk, 0)
    pltpu.sync_copy(stage_v, out.at[pl.ds(wid * 16, 16)])


@functools.partial(
    pl.kernel,
    out_type=jax.ShapeDtypeStruct((N,), jnp.float32),
    mesh=_mesh,
    scratch_types=[
        pltpu.VMEM((IDXROWS, LANE), jnp.int32),
        pltpu.VMEM((IDXROWS, LANE), jnp.int32),
        pltpu.VMEM((LANE, LANE), jnp.float32),
        pltpu.VMEM((LANE, LANE), jnp.float32),
        pltpu.VMEM((B // 8, LANE), jnp.float32),
        pltpu.VMEM((B // LANE, LANE), jnp.int32),
        pltpu.VMEM((RPW,), jnp.float32),
        pltpu.SemaphoreType.DMA,
    ],
    compiler_params=pltpu.CompilerParams(needs_layout_passes=False),
)
def _sc_scores(emb_u, ug, us, cent, invc, out,
               ug_v, usub_v, ubufa, ubufb, cent_v, invc_v,
               scores_v, usem):
    wid = lax.axis_index("s") * 2 + lax.axis_index("c")

    pltpu.sync_copy(ug.at[wid], ug_v)
    pltpu.sync_copy(us.at[wid], usub_v)
    pltpu.async_copy(emb_u.at[ug_v.at[0]], ubufa, usem)
    pltpu.sync_copy(cent, cent_v)
    pltpu.sync_copy(invc, invc_v)

    iota16 = lax.iota(jnp.int32, 16)

    def score_chunk(j, buf):
        def blk(b, carry):
            row = j * LANE + b * 16
            svec = usub_v[row // LANE, pl.ds(row % LANE, 16)]
            slot = b * 16 + iota16
            rvec = row + iota16
            bvec = jnp.where(rvec < BPW, rvec, (rvec - BPW) // NEG)
            gb = wid * BPW + bvec
            pcv = plsc.load_gather(invc_v, [gb // LANE, gb % LANE])
            pg = pcv // 8
            plane = (pcv % 8) * DIM
            acc = jnp.zeros((16,), jnp.float32)
            for d in range(DIM):
                uc = plsc.load_gather(buf, [slot, svec + d])
                vc = plsc.load_gather(cent_v, [pg, plane + d])
                acc = acc + uc * vc
            scores_v[pl.ds(row, 16)] = acc
            return carry

        lax.fori_loop(0, LANE // 16, blk, 0)

    def u_chunk(j, carry):
        even = j % 2 == 0

        @pl.when(jnp.logical_and(j + 1 < NCH, even))
        def _():
            pltpu.async_copy(emb_u.at[ug_v.at[j + 1]], ubufb, usem)

        @pl.when(jnp.logical_and(j + 1 < NCH, jnp.logical_not(even)))
        def _():
            pltpu.async_copy(emb_u.at[ug_v.at[j + 1]], ubufa, usem)

        @pl.when(even)
        def _():
            pltpu.make_async_copy(emb_u.at[ug_v.at[j]], ubufa, usem).wait()
            score_chunk(j, ubufa)

        @pl.when(jnp.logical_not(even))
        def _():
            pltpu.make_async_copy(emb_u.at[ug_v.at[j]], ubufb, usem).wait()
            score_chunk(j, ubufb)

        return carry

    lax.fori_loop(0, NCH, u_chunk, 0)

    pltpu.sync_copy(scores_v.at[pl.ds(0, BPW)], out.at[pl.ds(wid * BPW, BPW)])
    pltpu.sync_copy(scores_v.at[pl.ds(BPW, RPW - BPW)],
                    out.at[pl.ds(B + wid * (RPW - BPW), RPW - BPW)])


def _tc_reduce_body(s_ref, o_ref):
    s = s_ref[...]
    ridx = lax.broadcasted_iota(jnp.int32, (ROWS2D, LANE), 0)
    t = jnp.where(ridx < POS_ROWS, s, -s)
    ls = jnp.minimum(t, 0.0) - jnp.log1p(jnp.exp(-jnp.abs(t)))
    o_ref[0, 0] = -jnp.sum(ls) / B


_tc_reduce = pl.pallas_call(
    _tc_reduce_body,
    out_shape=jax.ShapeDtypeStruct((1, 1), jnp.float32),
    out_specs=pl.BlockSpec(memory_space=pltpu.SMEM),
)


def _pad_rows(x2d, rows):
    out = jnp.zeros((NW, rows * LANE), x2d.dtype)
    out = lax.dynamic_update_slice(out, x2d, (0, 0))
    return out.reshape(NW, rows, LANE)


def kernel(embedding_v, embedding_u, center_words, target_words, negative_words):
    c = center_words.reshape(-1).astype(jnp.int32)
    t = target_words.reshape(-1).astype(jnp.int32)
    n = negative_words.reshape(-1).astype(jnp.int32)
    uidx = jnp.concatenate(
        [t.reshape(NW, BPW), n.reshape(NW, RPW - BPW)], axis=1)
    ug = _pad_rows(uidx // 8, IDXROWS)
    us = _pad_rows((uidx % 8) * DIM, IDXROWS)
    sort_c, ord_c = lax.sort_key_val(c, lax.iota(jnp.int32, B))
    inv_c = jnp.zeros((B,), jnp.int32).at[ord_c].set(lax.iota(jnp.int32, B))
    emb_v_t = embedding_v.T
    v_tail = emb_v_t[:, TB:]
    cent = _build_cent(emb_v_t, v_tail, sort_c.reshape(NW, 1, LANE))
    emb_u2 = embedding_u.reshape(VOCAB // 8, 8 * DIM)
    scores = _sc_scores(emb_u2, ug, us, cent, inv_c.reshape(B // LANE, LANE))
    loss = _tc_reduce(scores.reshape(ROWS2D, LANE))
    return loss[0, 0]
